# Initial kernel scaffold; baseline (speedup 1.0000x reference)
import jax
import jax.numpy as jnp
from jax.experimental import pallas as pl


def kernel(wi, bi, wh, bhn, w1, b1, w2, b2, state, gru_hx):
    raise NotImplementedError("write your pallas kernel here")


# speedup vs baseline: 1.1367x; 1.1367x over previous
import functools

import jax
import jax.numpy as jnp
import numpy as np
from jax.experimental import pallas as pl
from jax.experimental.pallas import tpu as pltpu

GP = 128
OUT_SIZE = 64


def _gru_ac_kernel(x_ref, Wc_ref, bc_ref, h0_ref, w1_ref, b1_ref, w2_ref,
                   b2_ref, out_ref, hfin_ref, go_scr):
    T, Bb, _ = x_ref.shape
    Wc = Wc_ref[...]
    bias = bc_ref[...]

    def step(t, h):
        lhs = jnp.concatenate([x_ref[t], h.astype(jnp.bfloat16)], axis=1)
        g = jnp.dot(lhs, Wc, preferred_element_type=jnp.float32) + bias
        rz = jax.nn.sigmoid(g[:, :2 * GP])
        r = rz[:, :GP]
        z = rz[:, GP:2 * GP]
        n = jnp.tanh(g[:, 2 * GP:3 * GP] + r * g[:, 3 * GP:])
        h_new = n + z * (h - n)
        go_scr[t] = h_new.astype(jnp.bfloat16)
        return h_new

    h_fin = jax.lax.fori_loop(0, T, step, h0_ref[...], unroll=8)
    hfin_ref[...] = h_fin

    w1 = w1_ref[...]
    b1 = b1_ref[...]
    w2 = w2_ref[...]
    b2 = b2_ref[...]
    TC = min(16, T)
    for c in range(T // TC):
        gch = go_scr[c * TC:(c + 1) * TC].reshape(TC * Bb, GP)
        h1 = jnp.maximum(
            jnp.dot(gch, w1, preferred_element_type=jnp.float32) + b1, 0.0)
        o = jnp.dot(h1.astype(jnp.bfloat16), w2,
                    preferred_element_type=jnp.float32) + b2
        out_ref[c * TC:(c + 1) * TC] = o.reshape(TC, Bb, GP)


@functools.partial(jax.jit, static_argnames=())
def kernel(wi, bi, wh, bhn, w1, b1, w2, b2, state, gru_hx):
    B, T, D = state.shape
    H = gru_hx.shape[-1]

    Wc = jnp.zeros((2 * GP, 4 * GP), jnp.float32)
    Wc = Wc.at[:D, :3 * GP].set(wi)
    Wc = Wc.at[GP:, :2 * GP].set(wh[:, :2 * GP])
    Wc = Wc.at[GP:, 3 * GP:].set(wh[:, 2 * GP:])
    Wc = Wc.astype(jnp.bfloat16)
    bc = jnp.concatenate([bi, bhn], axis=1)

    B_pad = ((B + 7) // 8) * 8
    B_blk = min(B_pad, 256)
    B_pad = ((B_pad + B_blk - 1) // B_blk) * B_blk
    nb = B_pad // B_blk

    x = jnp.zeros((B_pad, T, D), jnp.float32).at[:B].set(state)
    x_tb = jnp.transpose(x, (1, 0, 2)).astype(jnp.bfloat16)
    h0 = jnp.zeros((B_pad, GP), jnp.float32).at[:B, :H].set(gru_hx[0])

    w1b = w1.astype(jnp.bfloat16)
    w2b = w2.astype(jnp.bfloat16)

    flops = (2 * T * B_pad * 2 * GP * 4 * GP
             + 2 * T * B_pad * GP * 2 * H
             + 2 * T * B_pad * 2 * H * GP)
    transcendentals = T * B_pad * 3 * GP

    def f32b(shape):
        return int(np.prod(shape)) * 4

    bytes_accessed = (f32b((T, B_pad, D)) // 2 + f32b((B_pad, GP))
                      + f32b((T, B_pad, GP)) + f32b((B_pad, GP)))

    out, hfin = pl.pallas_call(
        _gru_ac_kernel,
        out_shape=(jax.ShapeDtypeStruct((T, B_pad, GP), jnp.float32),
                   jax.ShapeDtypeStruct((B_pad, GP), jnp.float32)),
        grid_spec=pltpu.PrefetchScalarGridSpec(
            num_scalar_prefetch=0,
            grid=(nb,),
            in_specs=[
                pl.BlockSpec((T, B_blk, D), lambda b: (0, b, 0)),
                pl.BlockSpec((2 * GP, 4 * GP), lambda b: (0, 0)),
                pl.BlockSpec((1, 4 * GP), lambda b: (0, 0)),
                pl.BlockSpec((B_blk, GP), lambda b: (b, 0)),
                pl.BlockSpec((GP, 2 * H), lambda b: (0, 0)),
                pl.BlockSpec((1, 2 * H), lambda b: (0, 0)),
                pl.BlockSpec((2 * H, GP), lambda b: (0, 0)),
                pl.BlockSpec((1, GP), lambda b: (0, 0)),
            ],
            out_specs=(
                pl.BlockSpec((T, B_blk, GP), lambda b: (0, b, 0)),
                pl.BlockSpec((B_blk, GP), lambda b: (b, 0)),
            ),
            scratch_shapes=[pltpu.VMEM((T, B_blk, GP), jnp.bfloat16)],
        ),
        compiler_params=pltpu.CompilerParams(
            dimension_semantics=("parallel",),
            vmem_limit_bytes=100 * 1024 * 1024),
        cost_estimate=pl.CostEstimate(flops=flops,
                                      transcendentals=transcendentals,
                                      bytes_accessed=bytes_accessed),
    )(x_tb, Wc, bc, h0, w1b, b1, w2b, b2)

    pol = jnp.transpose(out[:, :B, :OUT_SIZE], (1, 0, 2))
    val = jnp.transpose(out[:, :B, OUT_SIZE:OUT_SIZE + 1], (1, 0, 2))
    return pol, val, hfin[:B, :H][None]


# speedup vs baseline: 1.5447x; 1.3589x over previous
import functools

import jax
import jax.numpy as jnp
import numpy as np
from jax.experimental import pallas as pl
from jax.experimental.pallas import tpu as pltpu

GP = 128
OUT_SIZE = 64


def _gru_ac_kernel(x_ref, Wc_ref, bc_ref, h0_ref, w1_ref, b1_ref, w2_ref,
                   b2_ref, pol_ref, val_ref, hfin_ref, xt_scr, go_scr):
    Bb, T, _ = x_ref.shape
    xt_scr[...] = jnp.transpose(x_ref[...], (1, 0, 2)).astype(jnp.bfloat16)
    Wc = Wc_ref[...]
    bias = bc_ref[...]

    def step(t, h):
        lhs = jnp.concatenate([xt_scr[t], h.astype(jnp.bfloat16)], axis=1)
        g = jnp.dot(lhs, Wc, preferred_element_type=jnp.float32) + bias
        rz = jax.nn.sigmoid(g[:, :2 * GP])
        r = rz[:, :GP]
        z = rz[:, GP:2 * GP]
        n = jnp.tanh(g[:, 2 * GP:3 * GP] + r * g[:, 3 * GP:])
        h_new = n + z * (h - n)
        go_scr[t] = h_new.astype(jnp.bfloat16)
        return h_new

    h_fin = jax.lax.fori_loop(0, T, step, h0_ref[0], unroll=8)
    hfin_ref[...] = h_fin

    w1 = w1_ref[...]
    b1 = b1_ref[...]
    w2 = w2_ref[...]
    b2 = b2_ref[...]
    TC = min(16, T)
    for c in range(T // TC):
        gch = go_scr[c * TC:(c + 1) * TC].reshape(TC * Bb, GP)
        h1 = jnp.maximum(
            jnp.dot(gch, w1, preferred_element_type=jnp.float32) + b1, 0.0)
        o = jnp.dot(h1.astype(jnp.bfloat16), w2,
                    preferred_element_type=jnp.float32) + b2
        ot = jnp.transpose(o.reshape(TC, Bb, GP), (1, 0, 2))
        pol_ref[:, c * TC:(c + 1) * TC, :] = ot[:, :, :OUT_SIZE]
        val_ref[:, c * TC:(c + 1) * TC, :] = ot[:, :, OUT_SIZE:OUT_SIZE + 1]


@functools.partial(jax.jit, static_argnames=())
def kernel(wi, bi, wh, bhn, w1, b1, w2, b2, state, gru_hx):
    B, T, D = state.shape
    H = gru_hx.shape[-1]

    Wc = jnp.zeros((2 * GP, 4 * GP), jnp.float32)
    Wc = Wc.at[:D, :3 * GP].set(wi)
    Wc = Wc.at[GP:, :2 * GP].set(wh[:, :2 * GP])
    Wc = Wc.at[GP:, 3 * GP:].set(wh[:, 2 * GP:])
    Wc = Wc.astype(jnp.bfloat16)
    bc = jnp.concatenate([bi, bhn], axis=1)

    B_blk = min(B, 256)
    nb = B // B_blk

    w1b = w1.astype(jnp.bfloat16)
    w2b = w2.astype(jnp.bfloat16)

    flops = (2 * T * B * 2 * GP * 4 * GP
             + 2 * T * B * GP * 2 * H
             + 2 * T * B * 2 * H * GP)
    transcendentals = T * B * 3 * GP

    def f32b(shape):
        return int(np.prod(shape)) * 4

    bytes_accessed = (f32b((T, B, D)) + f32b((B, GP))
                      + f32b((T, B, GP)) + f32b((B, GP)))

    pol, val, hfin = pl.pallas_call(
        _gru_ac_kernel,
        out_shape=(jax.ShapeDtypeStruct((B, T, OUT_SIZE), jnp.float32),
                   jax.ShapeDtypeStruct((B, T, 1), jnp.float32),
                   jax.ShapeDtypeStruct((B, GP), jnp.float32)),
        grid_spec=pltpu.PrefetchScalarGridSpec(
            num_scalar_prefetch=0,
            grid=(nb,),
            in_specs=[
                pl.BlockSpec((B_blk, T, D), lambda b: (b, 0, 0)),
                pl.BlockSpec((2 * GP, 4 * GP), lambda b: (0, 0)),
                pl.BlockSpec((1, 4 * GP), lambda b: (0, 0)),
                pl.BlockSpec((1, B_blk, GP), lambda b: (0, b, 0)),
                pl.BlockSpec((GP, 2 * H), lambda b: (0, 0)),
                pl.BlockSpec((1, 2 * H), lambda b: (0, 0)),
                pl.BlockSpec((2 * H, GP), lambda b: (0, 0)),
                pl.BlockSpec((1, GP), lambda b: (0, 0)),
            ],
            out_specs=(
                pl.BlockSpec((B_blk, T, OUT_SIZE), lambda b: (b, 0, 0)),
                pl.BlockSpec((B_blk, T, 1), lambda b: (b, 0, 0)),
                pl.BlockSpec((B_blk, GP), lambda b: (b, 0)),
            ),
            scratch_shapes=[pltpu.VMEM((T, B_blk, D), jnp.bfloat16),
                            pltpu.VMEM((T, B_blk, GP), jnp.bfloat16)],
        ),
        compiler_params=pltpu.CompilerParams(
            dimension_semantics=("parallel",),
            vmem_limit_bytes=60 * 1024 * 1024),
        cost_estimate=pl.CostEstimate(flops=flops,
                                      transcendentals=transcendentals,
                                      bytes_accessed=bytes_accessed),
    )(state, Wc, bc, gru_hx, w1b, b1, w2b, b2)

    return pol, val, hfin[None, :, :H]


# speedup vs baseline: 1.7768x; 1.1502x over previous
import functools

import jax
import jax.numpy as jnp
import numpy as np
from jax.experimental import pallas as pl
from jax.experimental.pallas import tpu as pltpu

GP = 128
OUT_SIZE = 64
TCHUNK = 16


def _gru_ac_kernel(x_ref, wi_ref, wh_ref, bi_ref, bhn_ref, h0_ref,
                   w1_ref, b1_ref, w2_ref, b2_ref,
                   pol_ref, val_ref, hfin_ref,
                   xt_scr, go_scr, h_scr, Wc_scr, bc_scr, w1_scr, w2_scr):
    Bb, Tc, _ = x_ref.shape
    ct = pl.program_id(1)
    n_ct = pl.num_programs(1)

    @pl.when(ct == 0)
    def _init():
        Wc_scr[:GP, :3 * GP] = wi_ref[...].astype(jnp.bfloat16)
        Wc_scr[:GP, 3 * GP:] = jnp.zeros((GP, GP), jnp.bfloat16)
        Wc_scr[GP:, :2 * GP] = wh_ref[:, :2 * GP].astype(jnp.bfloat16)
        Wc_scr[GP:, 2 * GP:3 * GP] = jnp.zeros((GP, GP), jnp.bfloat16)
        Wc_scr[GP:, 3 * GP:] = wh_ref[:, 2 * GP:].astype(jnp.bfloat16)
        bc_scr[0:1, :3 * GP] = bi_ref[...]
        bc_scr[0:1, 3 * GP:] = bhn_ref[...]
        w1_scr[...] = w1_ref[...].astype(jnp.bfloat16)
        w2_scr[...] = w2_ref[...].astype(jnp.bfloat16)
        h_scr[...] = h0_ref[0]

    xt_scr[...] = jnp.transpose(x_ref[...], (1, 0, 2)).astype(jnp.bfloat16)
    Wc = Wc_scr[...]
    bias = bc_scr[...]

    def step(t, h):
        lhs = jnp.concatenate([xt_scr[t], h.astype(jnp.bfloat16)], axis=1)
        g = jnp.dot(lhs, Wc, preferred_element_type=jnp.float32) + bias
        rz = jax.nn.sigmoid(g[:, :2 * GP])
        r = rz[:, :GP]
        z = rz[:, GP:2 * GP]
        n = jnp.tanh(g[:, 2 * GP:3 * GP] + r * g[:, 3 * GP:])
        h_new = n + z * (h - n)
        go_scr[t] = h_new.astype(jnp.bfloat16)
        return h_new

    h_fin = jax.lax.fori_loop(0, Tc, step, h_scr[...], unroll=8)
    h_scr[...] = h_fin

    @pl.when(ct == n_ct - 1)
    def _fin():
        hfin_ref[...] = h_fin

    gch = go_scr[...].reshape(Tc * Bb, GP)
    h1 = jnp.maximum(
        jnp.dot(gch, w1_scr[...], preferred_element_type=jnp.float32)
        + b1_ref[...], 0.0)
    o = jnp.dot(h1.astype(jnp.bfloat16), w2_scr[...],
                preferred_element_type=jnp.float32) + b2_ref[...]
    ot = jnp.transpose(o.reshape(Tc, Bb, GP), (1, 0, 2))
    pol_ref[...] = ot[:, :, :OUT_SIZE]
    val_ref[...] = ot[:, :, OUT_SIZE:OUT_SIZE + 1]


@functools.partial(jax.jit, static_argnames=())
def kernel(wi, bi, wh, bhn, w1, b1, w2, b2, state, gru_hx):
    B, T, D = state.shape
    H = gru_hx.shape[-1]

    B_blk = min(B, 256)
    nb = B // B_blk
    Tc = min(TCHUNK, T)
    nT = T // Tc

    flops = (2 * T * B * 2 * GP * 4 * GP
             + 2 * T * B * GP * 2 * H
             + 2 * T * B * 2 * H * GP)
    transcendentals = T * B * 3 * GP

    def f32b(shape):
        return int(np.prod(shape)) * 4

    bytes_accessed = (f32b((T, B, D)) + f32b((B, GP))
                      + f32b((T, B, GP)) + f32b((B, GP)))

    pol, val, hfin = pl.pallas_call(
        _gru_ac_kernel,
        out_shape=(jax.ShapeDtypeStruct((B, T, OUT_SIZE), jnp.float32),
                   jax.ShapeDtypeStruct((B, T, 1), jnp.float32),
                   jax.ShapeDtypeStruct((B, GP), jnp.float32)),
        grid_spec=pltpu.PrefetchScalarGridSpec(
            num_scalar_prefetch=0,
            grid=(nb, nT),
            in_specs=[
                pl.BlockSpec((B_blk, Tc, D), lambda b, t: (b, t, 0)),
                pl.BlockSpec((D, 3 * GP), lambda b, t: (0, 0)),
                pl.BlockSpec((GP, 3 * GP), lambda b, t: (0, 0)),
                pl.BlockSpec((1, 3 * GP), lambda b, t: (0, 0)),
                pl.BlockSpec((1, GP), lambda b, t: (0, 0)),
                pl.BlockSpec((1, B_blk, GP), lambda b, t: (0, b, 0)),
                pl.BlockSpec((GP, 2 * H), lambda b, t: (0, 0)),
                pl.BlockSpec((1, 2 * H), lambda b, t: (0, 0)),
                pl.BlockSpec((2 * H, GP), lambda b, t: (0, 0)),
                pl.BlockSpec((1, GP), lambda b, t: (0, 0)),
            ],
            out_specs=(
                pl.BlockSpec((B_blk, Tc, OUT_SIZE), lambda b, t: (b, t, 0)),
                pl.BlockSpec((B_blk, Tc, 1), lambda b, t: (b, t, 0)),
                pl.BlockSpec((B_blk, GP), lambda b, t: (b, 0)),
            ),
            scratch_shapes=[pltpu.VMEM((Tc, B_blk, D), jnp.bfloat16),
                            pltpu.VMEM((Tc, B_blk, GP), jnp.bfloat16),
                            pltpu.VMEM((B_blk, GP), jnp.float32),
                            pltpu.VMEM((2 * GP, 4 * GP), jnp.bfloat16),
                            pltpu.VMEM((1, 4 * GP), jnp.float32),
                            pltpu.VMEM((GP, 2 * H), jnp.bfloat16),
                            pltpu.VMEM((2 * H, GP), jnp.bfloat16)],
        ),
        compiler_params=pltpu.CompilerParams(
            dimension_semantics=("parallel", "arbitrary"),
            vmem_limit_bytes=60 * 1024 * 1024),
        cost_estimate=pl.CostEstimate(flops=flops,
                                      transcendentals=transcendentals,
                                      bytes_accessed=bytes_accessed),
    )(state, wi, wh, bi, bhn, gru_hx, w1, b1, w2, b2)

    return pol, val, hfin[None, :, :H]
